# baseline (speedup 1.0000x reference)
import functools
import math

import jax
import jax.numpy as jnp
from jax import lax
from jax.experimental import pallas as pl
from jax.experimental.pallas import tpu as pltpu
from jax.experimental.pallas import tpu_sc as plsc

_NC = 2
_NS = 16
_NW = _NC * _NS
_L = 16


def _emb_kernel_factory(J, IB, D, scale):
    mesh = plsc.VectorSubcoreMesh(core_axis_name="c", subcore_axis_name="s")
    DB = D // 8
    KD = D // _L

    @functools.partial(
        pl.kernel,
        mesh=mesh,
        out_type=jax.ShapeDtypeStruct((J, DB, _NW, 8, IB), jnp.float32),
        scratch_types=[
            pltpu.VMEM((J, IB), jnp.int32),
            pltpu.VMEM((IB, 2 * D), jnp.float32),
            pltpu.VMEM((IB, 2 * D), jnp.float32),
            pltpu.VMEM((IB, 2 * D), jnp.float32),
            pltpu.VMEM((IB, 2 * D), jnp.float32),
            pltpu.VMEM((DB, 8, IB), jnp.float32),
            pltpu.VMEM((DB, 8, IB), jnp.float32),
            pltpu.SemaphoreType.DMA,
            pltpu.SemaphoreType.DMA,
            pltpu.SemaphoreType.DMA,
            pltpu.SemaphoreType.DMA,
            pltpu.SemaphoreType.DMA,
            pltpu.SemaphoreType.DMA,
        ],
        compiler_params=pltpu.CompilerParams(needs_layout_passes=False),
    )
    def emb(xt_hbm, lut_hbm, out_hbm, idx_v, gb0, gb1, gb2, gb3, ob0, ob1,
            gs0, gs1, gs2, gs3, os0, os1):
        wid = lax.axis_index("s") * _NC + lax.axis_index("c")
        gbufs, obufs = (gb0, gb1, gb2, gb3), (ob0, ob1)
        gsems, osems = (gs0, gs1, gs2, gs3), (os0, os1)
        pltpu.sync_copy(xt_hbm.at[:, pl.ds(wid * IB, IB)], idx_v)

        iota = lax.iota(jnp.int32, _L)
        zero16 = iota * 0
        i0k = [(lax.shift_right_logical(iota, 3) + 2 * k) for k in range(KD)]
        i1 = iota & 7

        def start_gather(j, b):
            pltpu.async_copy(lut_hbm.at[idx_v.at[j]], gbufs[b], gsems[b])

        def wait_gather(b):
            pltpu.make_async_copy(
                lut_hbm.at[pl.ds(0, IB)], gbufs[b], gsems[b]
            ).wait()

        def wait_out(b):
            pltpu.make_async_copy(
                out_hbm.at[0, :, wid], obufs[b], osems[b]
            ).wait()

        for q in range(4):
            start_gather(q, q)

        def quad_body(jj, c):
            for q in range(4):
                j = 4 * jj + q
                gbuf, obuf = gbufs[q], obufs[q % 2]

                wait_gather(q)

                @pl.when(j >= 2)
                def _():
                    wait_out(q % 2)

                @plsc.parallel_loop(0, IB, step=1, unroll=8)
                def row_body(r):
                    r16 = zero16 + r
                    for k in range(KD):
                        v = gbuf[r, pl.ds(k * _L, _L)]
                        plsc.store_scatter(obuf, [i0k[k], i1, r16], v)

                @pl.when(j + 4 < J)
                def _():
                    start_gather(j + 4, q)

                pltpu.async_copy(obuf, out_hbm.at[j, :, wid], osems[q % 2])
            return c

        lax.fori_loop(0, J // 4, quad_body, 0)
        wait_out(0)
        wait_out(1)

    return emb


def kernel(x, lut):
    B0, B1 = x.shape
    V, D = lut.shape
    scale = float(math.sqrt(D))
    IB = B0 // _NW
    lutp = jnp.pad(lut * scale, ((0, 0), (0, D)))
    xt = x.T
    out5 = _emb_kernel_factory(B1, IB, D, scale)(xt, lutp)
    return out5.transpose(2, 4, 0, 1, 3).reshape(B0, B1, D)


# speedup vs baseline: 1.2597x; 1.2597x over previous
import functools
import math

import jax
import jax.numpy as jnp
from jax import lax
from jax.experimental import pallas as pl
from jax.experimental.pallas import tpu as pltpu
from jax.experimental.pallas import tpu_sc as plsc

_NC = 2
_NS = 16
_NW = _NC * _NS
_L = 16


def _emb_kernel_factory(J, IB, D, scale):
    mesh = plsc.VectorSubcoreMesh(core_axis_name="c", subcore_axis_name="s")
    DB = D // 8
    KD = D // _L

    @functools.partial(
        pl.kernel,
        mesh=mesh,
        out_type=jax.ShapeDtypeStruct((J, DB, _NW, 8, IB), jnp.float32),
        scratch_types=[
            pltpu.VMEM((J, IB), jnp.int32),
            pltpu.VMEM((IB, D), jnp.float32),
            pltpu.VMEM((IB, D), jnp.float32),
            pltpu.VMEM((IB, D), jnp.float32),
            pltpu.VMEM((IB, D), jnp.float32),
            pltpu.VMEM((DB, 8, IB), jnp.float32),
            pltpu.VMEM((DB, 8, IB), jnp.float32),
            pltpu.SemaphoreType.DMA,
            pltpu.SemaphoreType.DMA,
            pltpu.SemaphoreType.DMA,
            pltpu.SemaphoreType.DMA,
            pltpu.SemaphoreType.DMA,
            pltpu.SemaphoreType.DMA,
        ],
        compiler_params=pltpu.CompilerParams(
            needs_layout_passes=False, use_tc_tiling_on_sc=False
        ),
    )
    def emb(xt_hbm, lut_hbm, out_hbm, idx_v, gb0, gb1, gb2, gb3, ob0, ob1,
            gs0, gs1, gs2, gs3, os0, os1):
        wid = lax.axis_index("s") * _NC + lax.axis_index("c")
        gbufs, obufs = (gb0, gb1, gb2, gb3), (ob0, ob1)
        gsems, osems = (gs0, gs1, gs2, gs3), (os0, os1)
        pltpu.sync_copy(xt_hbm.at[:, pl.ds(wid * IB, IB)], idx_v)

        iota = lax.iota(jnp.int32, _L)
        zero16 = iota * 0
        i0k = [(lax.shift_right_logical(iota, 3) + 2 * k) for k in range(KD)]
        i1 = iota & 7

        def start_gather(j, b):
            pltpu.async_copy(lut_hbm.at[idx_v.at[j]], gbufs[b], gsems[b])

        def wait_gather(b):
            pltpu.make_async_copy(
                lut_hbm.at[pl.ds(0, IB)], gbufs[b], gsems[b]
            ).wait()

        def wait_out(b):
            pltpu.make_async_copy(
                out_hbm.at[0, :, wid], obufs[b], osems[b]
            ).wait()

        for q in range(4):
            start_gather(q, q)

        def quad_body(jj, c):
            for q in range(4):
                j = 4 * jj + q
                gbuf, obuf = gbufs[q], obufs[q % 2]

                wait_gather(q)

                @pl.when(j >= 2)
                def _():
                    wait_out(q % 2)

                @plsc.parallel_loop(0, IB, step=1, unroll=8)
                def row_body(r):
                    r16 = zero16 + r
                    for k in range(KD):
                        v = gbuf[r, pl.ds(k * _L, _L)]
                        plsc.store_scatter(obuf, [i0k[k], i1, r16], v)

                @pl.when(j + 4 < J)
                def _():
                    start_gather(j + 4, q)

                pltpu.async_copy(obuf, out_hbm.at[j, :, wid], osems[q % 2])
            return c

        lax.fori_loop(0, J // 4, quad_body, 0)
        wait_out(0)
        wait_out(1)

    return emb


def kernel(x, lut):
    B0, B1 = x.shape
    V, D = lut.shape
    scale = float(math.sqrt(D))
    IB = B0 // _NW
    lutp = (jnp.pad(lut, ((0, 0), (0, D))) * scale).reshape(2 * V, D)
    xt = x.T * 2
    out5 = _emb_kernel_factory(B1, IB, D, scale)(xt, lutp)
    return out5.transpose(2, 4, 0, 1, 3).reshape(B0, B1, D)
